# Initial kernel scaffold; baseline (speedup 1.0000x reference)
import jax
import jax.numpy as jnp
from jax.experimental import pallas as pl


def kernel(x, W, b, gamma, beta):
    raise NotImplementedError("write your pallas kernel here")


# speedup vs baseline: 5.1722x; 5.1722x over previous
import jax
import jax.numpy as jnp
from jax import lax
from jax.experimental import pallas as pl

K = 20
TOPK_TN = 256
COMB_TN = 128


def _topk_kernel(xt_ref, x_ref, w1_ref, w2_ref, b_ref,
                 idx_ref, w_ref, y1_ref, y2_ref):
    xt = xt_ref[0]
    xb = x_ref[0]
    s_row = jnp.sum(xt * xt, axis=1, keepdims=True)
    s_col = jnp.sum(xb * xb, axis=0, keepdims=True)
    prod = jnp.dot(xt, xb, preferred_element_type=jnp.float32,
                   precision=lax.Precision.HIGHEST)
    d = (-2.0 * prod + s_row) + s_col
    tn, n = d.shape
    col = lax.broadcasted_iota(jnp.int32, (tn, n), 1)
    idx_cols = []
    w_cols = []
    for t in range(K + 1):
        m = jnp.min(d, axis=1, keepdims=True)
        cand = jnp.where(d == m, col, jnp.int32(n))
        i_t = jnp.min(cand, axis=1, keepdims=True)
        if t > 0:
            idx_cols.append(i_t)
            dist_sat = 2.0 * jax.nn.sigmoid(m) - 1.0
            w_cols.append(1.0 - dist_sat)
        d = jnp.where(col == i_t, jnp.float32(jnp.inf), d)
    idx_ref[0] = jnp.concatenate(idx_cols, axis=1)
    w_ref[0] = jnp.concatenate(w_cols, axis=1)
    y1_ref[0] = jnp.dot(xt, w1_ref[...], preferred_element_type=jnp.float32,
                        precision=lax.Precision.HIGHEST) + b_ref[...]
    y2_ref[0] = jnp.dot(xt, w2_ref[...], preferred_element_type=jnp.float32,
                        precision=lax.Precision.HIGHEST)


def _combine_kernel(idx_ref, w_ref, y1_ref, y2_ref,
                    mx_ref, mn_ref, s1_ref, s2_ref):
    idx = idx_ref[0]
    wv = w_ref[0]
    y1 = y1_ref[0]
    y2 = y2_ref[0]
    tn = idx.shape[0]
    n = y2.shape[0]
    col = lax.broadcasted_iota(jnp.int32, (tn, n), 1)
    mx = None
    mn = None
    s1 = jnp.zeros((1, y1.shape[1]), jnp.float32)
    s2 = jnp.zeros((1, y1.shape[1]), jnp.float32)
    for j in range(K):
        oh = (col == idx[:, j:j + 1]).astype(jnp.float32)
        g = jnp.dot(oh, y2, preferred_element_type=jnp.float32,
                    precision=lax.Precision.HIGHEST)
        o = y1 + wv[:, j:j + 1] * g
        mx = o if mx is None else jnp.maximum(mx, o)
        mn = o if mn is None else jnp.minimum(mn, o)
        s1 = s1 + jnp.sum(o, axis=0, keepdims=True)
        s2 = s2 + jnp.sum(o * o, axis=0, keepdims=True)
    mx_ref[0] = mx
    mn_ref[0] = mn
    step = pl.program_id(0) * pl.num_programs(1) + pl.program_id(1)

    @pl.when(step == 0)
    def _():
        s1_ref[...] = jnp.zeros_like(s1_ref)
        s2_ref[...] = jnp.zeros_like(s2_ref)

    s1_ref[...] += s1
    s2_ref[...] += s2


def _finalize_kernel(mx_ref, mn_ref, s1_ref, s2_ref, g_ref, be_ref, o_ref,
                     *, count):
    s1 = s1_ref[...]
    s2 = s2_ref[...]
    mean = s1 * (1.0 / count)
    var = s2 * (1.0 / count) - mean * mean
    gm = g_ref[...]
    scale = gm * lax.rsqrt(var + 1e-5)
    shift = be_ref[...] - mean * scale
    sel = jnp.where(gm >= 0.0, mx_ref[0], mn_ref[0])
    o_ref[0] = jnp.maximum(sel * scale + shift, 0.0)


def kernel(x, W, b, gamma, beta):
    B, Fin, N = x.shape
    Fout = W.shape[0]
    xt = jnp.transpose(x, (0, 2, 1))
    w1t = jnp.transpose(W[:, :Fin])
    w2t = jnp.transpose(W[:, Fin:])
    b2 = b.reshape(1, Fout)
    g2 = gamma.reshape(1, Fout)
    be2 = beta.reshape(1, Fout)

    tn = TOPK_TN
    idx, wgt, y1, y2 = pl.pallas_call(
        _topk_kernel,
        grid=(B, N // tn),
        in_specs=[
            pl.BlockSpec((1, tn, Fin), lambda bb, i: (bb, i, 0)),
            pl.BlockSpec((1, Fin, N), lambda bb, i: (bb, 0, 0)),
            pl.BlockSpec((Fin, Fout), lambda bb, i: (0, 0)),
            pl.BlockSpec((Fin, Fout), lambda bb, i: (0, 0)),
            pl.BlockSpec((1, Fout), lambda bb, i: (0, 0)),
        ],
        out_specs=[
            pl.BlockSpec((1, tn, K), lambda bb, i: (bb, i, 0)),
            pl.BlockSpec((1, tn, K), lambda bb, i: (bb, i, 0)),
            pl.BlockSpec((1, tn, Fout), lambda bb, i: (bb, i, 0)),
            pl.BlockSpec((1, tn, Fout), lambda bb, i: (bb, i, 0)),
        ],
        out_shape=[
            jax.ShapeDtypeStruct((B, N, K), jnp.int32),
            jax.ShapeDtypeStruct((B, N, K), jnp.float32),
            jax.ShapeDtypeStruct((B, N, Fout), jnp.float32),
            jax.ShapeDtypeStruct((B, N, Fout), jnp.float32),
        ],
    )(xt, x, w1t, w2t, b2)

    ctn = COMB_TN
    mx, mn, s1, s2 = pl.pallas_call(
        _combine_kernel,
        grid=(B, N // ctn),
        in_specs=[
            pl.BlockSpec((1, ctn, K), lambda bb, i: (bb, i, 0)),
            pl.BlockSpec((1, ctn, K), lambda bb, i: (bb, i, 0)),
            pl.BlockSpec((1, ctn, Fout), lambda bb, i: (bb, i, 0)),
            pl.BlockSpec((1, N, Fout), lambda bb, i: (bb, 0, 0)),
        ],
        out_specs=[
            pl.BlockSpec((1, ctn, Fout), lambda bb, i: (bb, i, 0)),
            pl.BlockSpec((1, ctn, Fout), lambda bb, i: (bb, i, 0)),
            pl.BlockSpec((1, Fout), lambda bb, i: (0, 0)),
            pl.BlockSpec((1, Fout), lambda bb, i: (0, 0)),
        ],
        out_shape=[
            jax.ShapeDtypeStruct((B, N, Fout), jnp.float32),
            jax.ShapeDtypeStruct((B, N, Fout), jnp.float32),
            jax.ShapeDtypeStruct((1, Fout), jnp.float32),
            jax.ShapeDtypeStruct((1, Fout), jnp.float32),
        ],
    )(idx, wgt, y1, y2)

    import functools
    fin_k = functools.partial(_finalize_kernel, count=float(B * N * K))
    out = pl.pallas_call(
        fin_k,
        grid=(B, N // tn),
        in_specs=[
            pl.BlockSpec((1, tn, Fout), lambda bb, i: (bb, i, 0)),
            pl.BlockSpec((1, tn, Fout), lambda bb, i: (bb, i, 0)),
            pl.BlockSpec((1, Fout), lambda bb, i: (0, 0)),
            pl.BlockSpec((1, Fout), lambda bb, i: (0, 0)),
            pl.BlockSpec((1, Fout), lambda bb, i: (0, 0)),
            pl.BlockSpec((1, Fout), lambda bb, i: (0, 0)),
        ],
        out_specs=pl.BlockSpec((1, tn, Fout), lambda bb, i: (bb, i, 0)),
        out_shape=jax.ShapeDtypeStruct((B, N, Fout), jnp.float32),
    )(mx, mn, s1, s2, g2, be2)

    return jnp.transpose(out, (0, 2, 1))


# speedup vs baseline: 11.6005x; 2.2428x over previous
import functools

import jax
import jax.numpy as jnp
from jax import lax
from jax.experimental import pallas as pl
from jax.experimental.pallas import tpu as pltpu
from jax.experimental.pallas import tpu_sc as plsc

K = 20
TOPK_TN = 256
COMB_TN = 128
GATHER_CHUNK = 128


def _topk_kernel(xt_ref, x_ref, w1_ref, w2_ref, b_ref,
                 idx_ref, w_ref, y1_ref, y2_ref):
    xt = xt_ref[0]
    xb = x_ref[0]
    s_row = jnp.sum(xt * xt, axis=1, keepdims=True)
    s_col = jnp.sum(xb * xb, axis=0, keepdims=True)
    prod = jnp.dot(xt, xb, preferred_element_type=jnp.float32,
                   precision=lax.Precision.HIGHEST)
    d = (-2.0 * prod + s_row) + s_col
    tn, n = d.shape
    col = lax.broadcasted_iota(jnp.int32, (tn, n), 1)
    off = pl.program_id(0) * jnp.int32(n)
    idx_cols = []
    w_cols = []
    for t in range(K + 1):
        m = jnp.min(d, axis=1, keepdims=True)
        cand = jnp.where(d == m, col, jnp.int32(n))
        i_t = jnp.min(cand, axis=1, keepdims=True)
        if t > 0:
            idx_cols.append(i_t)
            dist_sat = 2.0 * jax.nn.sigmoid(m) - 1.0
            w_cols.append(1.0 - dist_sat)
        d = jnp.where(col == i_t, jnp.float32(jnp.inf), d)
    idx_ref[0] = jnp.concatenate(idx_cols, axis=1) + off
    w_ref[0] = jnp.concatenate(w_cols, axis=1)
    y1_ref[0] = jnp.dot(xt, w1_ref[...], preferred_element_type=jnp.float32,
                        precision=lax.Precision.HIGHEST) + b_ref[...]
    y2_ref[0] = jnp.dot(xt, w2_ref[...], preferred_element_type=jnp.float32,
                        precision=lax.Precision.HIGHEST)


def _make_sc_gather(rows_total, d_model):
    info = plsc.get_sparse_core_info()
    nc, ns = info.num_cores, info.num_subcores
    nw = nc * ns
    per_w = rows_total // nw
    ch = GATHER_CHUNK
    n_ch = per_w // ch
    assert per_w % ch == 0 and n_ch % 2 == 0
    mesh = plsc.VectorSubcoreMesh(core_axis_name="c", subcore_axis_name="s")

    @functools.partial(
        pl.kernel, mesh=mesh,
        compiler_params=pltpu.CompilerParams(use_tc_tiling_on_sc=False),
        out_type=jax.ShapeDtypeStruct((rows_total, d_model), jnp.float32),
        scratch_types=[
            pltpu.VMEM((per_w,), jnp.int32),
            pltpu.VMEM((ch, d_model), jnp.float32),
            pltpu.VMEM((ch, d_model), jnp.float32),
            pltpu.SemaphoreType.DMA,
            pltpu.SemaphoreType.DMA,
        ],
    )
    def sc_gather(table_hbm, idx_hbm, out_hbm, idx_v, rows0, rows1,
                  sem0, sem1):
        wid = lax.axis_index("s") * nc + lax.axis_index("c")
        base = pl.multiple_of(wid * per_w, ch)
        pltpu.sync_copy(idx_hbm.at[pl.ds(base, per_w)], idx_v)
        bufs = (rows0, rows1)
        sems = (sem0, sem1)

        def copy(i, slot):
            src = table_hbm.at[idx_v.at[pl.ds(pl.multiple_of(i * ch, ch), ch)]]
            return pltpu.make_async_copy(src, bufs[slot], sems[slot])

        copy(0, 0).start()
        copy(1, 1).start()

        def body(g, carry):
            i0 = g * 2
            for s in range(2):
                i = i0 + s
                copy(i, s).wait()
                pltpu.sync_copy(
                    bufs[s],
                    out_hbm.at[pl.ds(pl.multiple_of(base + i * ch, ch), ch)])

                @pl.when(i + 2 < n_ch)
                def _():
                    copy(i + 2, s).start()
            return carry

        lax.fori_loop(0, n_ch // 2, body, 0)

    return sc_gather


def _combine_kernel(g_ref, w_ref, y1_ref, mx_ref, mn_ref, s1_ref, s2_ref):
    wv = w_ref[0]
    y1 = y1_ref[0]
    mx = None
    mn = None
    s1 = jnp.zeros((1, y1.shape[1]), jnp.float32)
    s2 = jnp.zeros((1, y1.shape[1]), jnp.float32)
    for j in range(K):
        g = g_ref[0, :, j, :]
        o = y1 + wv[:, j:j + 1] * g
        mx = o if mx is None else jnp.maximum(mx, o)
        mn = o if mn is None else jnp.minimum(mn, o)
        s1 = s1 + jnp.sum(o, axis=0, keepdims=True)
        s2 = s2 + jnp.sum(o * o, axis=0, keepdims=True)
    mx_ref[0] = mx
    mn_ref[0] = mn
    step = pl.program_id(0) * pl.num_programs(1) + pl.program_id(1)

    @pl.when(step == 0)
    def _():
        s1_ref[...] = jnp.zeros_like(s1_ref)
        s2_ref[...] = jnp.zeros_like(s2_ref)

    s1_ref[...] += s1
    s2_ref[...] += s2


def _finalize_kernel(mx_ref, mn_ref, s1_ref, s2_ref, g_ref, be_ref, o_ref,
                     *, count):
    s1 = s1_ref[...]
    s2 = s2_ref[...]
    mean = s1 * (1.0 / count)
    var = s2 * (1.0 / count) - mean * mean
    gm = g_ref[...]
    scale = gm * lax.rsqrt(var + 1e-5)
    shift = be_ref[...] - mean * scale
    sel = jnp.where(gm >= 0.0, mx_ref[0], mn_ref[0])
    o_ref[0] = jnp.maximum(sel * scale + shift, 0.0)


def kernel(x, W, b, gamma, beta):
    B, Fin, N = x.shape
    Fout = W.shape[0]
    xt = jnp.transpose(x, (0, 2, 1))
    w1t = jnp.transpose(W[:, :Fin])
    w2t = jnp.transpose(W[:, Fin:])
    b2 = b.reshape(1, Fout)
    g2 = gamma.reshape(1, Fout)
    be2 = beta.reshape(1, Fout)

    tn = TOPK_TN
    idx, wgt, y1, y2 = pl.pallas_call(
        _topk_kernel,
        grid=(B, N // tn),
        in_specs=[
            pl.BlockSpec((1, tn, Fin), lambda bb, i: (bb, i, 0)),
            pl.BlockSpec((1, Fin, N), lambda bb, i: (bb, 0, 0)),
            pl.BlockSpec((Fin, Fout), lambda bb, i: (0, 0)),
            pl.BlockSpec((Fin, Fout), lambda bb, i: (0, 0)),
            pl.BlockSpec((1, Fout), lambda bb, i: (0, 0)),
        ],
        out_specs=[
            pl.BlockSpec((1, tn, K), lambda bb, i: (bb, i, 0)),
            pl.BlockSpec((1, tn, K), lambda bb, i: (bb, i, 0)),
            pl.BlockSpec((1, tn, Fout), lambda bb, i: (bb, i, 0)),
            pl.BlockSpec((1, tn, Fout), lambda bb, i: (bb, i, 0)),
        ],
        out_shape=[
            jax.ShapeDtypeStruct((B, N, K), jnp.int32),
            jax.ShapeDtypeStruct((B, N, K), jnp.float32),
            jax.ShapeDtypeStruct((B, N, Fout), jnp.float32),
            jax.ShapeDtypeStruct((B, N, Fout), jnp.float32),
        ],
    )(xt, x, w1t, w2t, b2)

    rows_total = B * N * K
    gath = _make_sc_gather(rows_total, Fout)(
        y2.reshape(B * N, Fout), idx.reshape(rows_total))
    gath = gath.reshape(B, N, K, Fout)

    ctn = COMB_TN
    mx, mn, s1, s2 = pl.pallas_call(
        _combine_kernel,
        grid=(B, N // ctn),
        in_specs=[
            pl.BlockSpec((1, ctn, K, Fout), lambda bb, i: (bb, i, 0, 0)),
            pl.BlockSpec((1, ctn, K), lambda bb, i: (bb, i, 0)),
            pl.BlockSpec((1, ctn, Fout), lambda bb, i: (bb, i, 0)),
        ],
        out_specs=[
            pl.BlockSpec((1, ctn, Fout), lambda bb, i: (bb, i, 0)),
            pl.BlockSpec((1, ctn, Fout), lambda bb, i: (bb, i, 0)),
            pl.BlockSpec((1, Fout), lambda bb, i: (0, 0)),
            pl.BlockSpec((1, Fout), lambda bb, i: (0, 0)),
        ],
        out_shape=[
            jax.ShapeDtypeStruct((B, N, Fout), jnp.float32),
            jax.ShapeDtypeStruct((B, N, Fout), jnp.float32),
            jax.ShapeDtypeStruct((1, Fout), jnp.float32),
            jax.ShapeDtypeStruct((1, Fout), jnp.float32),
        ],
    )(gath, wgt, y1)

    fin_k = functools.partial(_finalize_kernel, count=float(B * N * K))
    out = pl.pallas_call(
        fin_k,
        grid=(B, N // tn),
        in_specs=[
            pl.BlockSpec((1, tn, Fout), lambda bb, i: (bb, i, 0)),
            pl.BlockSpec((1, tn, Fout), lambda bb, i: (bb, i, 0)),
            pl.BlockSpec((1, Fout), lambda bb, i: (0, 0)),
            pl.BlockSpec((1, Fout), lambda bb, i: (0, 0)),
            pl.BlockSpec((1, Fout), lambda bb, i: (0, 0)),
            pl.BlockSpec((1, Fout), lambda bb, i: (0, 0)),
        ],
        out_specs=pl.BlockSpec((1, tn, Fout), lambda bb, i: (bb, i, 0)),
        out_shape=jax.ShapeDtypeStruct((B, N, Fout), jnp.float32),
    )(mx, mn, s1, s2, g2, be2)

    return jnp.transpose(out, (0, 2, 1))


# speedup vs baseline: 13.8135x; 1.1908x over previous
import functools

import jax
import jax.numpy as jnp
from jax import lax
from jax.experimental import pallas as pl
from jax.experimental.pallas import tpu as pltpu
from jax.experimental.pallas import tpu_sc as plsc

K = 20
TOPK_TN = 256
COMB_TN = 128
GATHER_CHUNK = 128


def _topk_kernel(xt_ref, x_ref, w1_ref, w2_ref, b_ref,
                 idx_ref, w_ref, y1_ref, y2_ref):
    xt = xt_ref[0]
    xb = x_ref[0]
    s_row = jnp.sum(xt * xt, axis=1, keepdims=True)
    s_col = jnp.sum(xb * xb, axis=0, keepdims=True)
    prod = jnp.dot(xt, xb, preferred_element_type=jnp.float32,
                   precision=lax.Precision.HIGHEST)
    d = (-2.0 * prod + s_row) + s_col
    tn, n = d.shape
    col = lax.broadcasted_iota(jnp.int32, (tn, n), 1)
    off = pl.program_id(0) * jnp.int32(n)
    bits = lax.bitcast_convert_type(d, jnp.int32)
    int_min = jnp.int32(-2147483648)
    mono = jnp.where(bits < 0, int_min - bits, bits)
    key = jnp.bitwise_or(jnp.bitwise_and(mono, jnp.int32(~2047)), col)
    idx_cols = []
    w_cols = []
    for t in range(K + 1):
        mk = jnp.min(key, axis=1, keepdims=True)
        if t > 0:
            idx_cols.append(jnp.bitwise_and(mk, jnp.int32(2047)))
            mono_m = jnp.bitwise_and(mk, jnp.int32(~2047))
            bits_m = jnp.where(mono_m < 0, int_min - mono_m, mono_m)
            m = lax.bitcast_convert_type(bits_m, jnp.float32)
            dist_sat = 2.0 * jax.nn.sigmoid(m) - 1.0
            w_cols.append(1.0 - dist_sat)
        key = jnp.where(key == mk, jnp.int32(2147483647), key)
    idx_ref[0] = jnp.concatenate(idx_cols, axis=1) + off
    w_ref[0] = jnp.concatenate(w_cols, axis=1)
    y1_ref[0] = jnp.dot(xt, w1_ref[...], preferred_element_type=jnp.float32,
                        precision=lax.Precision.HIGHEST) + b_ref[...]
    y2_ref[0] = jnp.dot(xt, w2_ref[...], preferred_element_type=jnp.float32,
                        precision=lax.Precision.HIGHEST)


def _make_sc_gather(rows_total, d_model):
    info = plsc.get_sparse_core_info()
    nc, ns = info.num_cores, info.num_subcores
    nw = nc * ns
    per_w = rows_total // nw
    ch = GATHER_CHUNK
    n_ch = per_w // ch
    assert per_w % ch == 0 and n_ch % 2 == 0
    mesh = plsc.VectorSubcoreMesh(core_axis_name="c", subcore_axis_name="s")

    @functools.partial(
        pl.kernel, mesh=mesh,
        compiler_params=pltpu.CompilerParams(use_tc_tiling_on_sc=False),
        out_type=jax.ShapeDtypeStruct((rows_total, d_model), jnp.float32),
        scratch_types=[
            pltpu.VMEM((per_w,), jnp.int32),
            pltpu.VMEM((ch, d_model), jnp.float32),
            pltpu.VMEM((ch, d_model), jnp.float32),
            pltpu.SemaphoreType.DMA,
            pltpu.SemaphoreType.DMA,
        ],
    )
    def sc_gather(table_hbm, idx_hbm, out_hbm, idx_v, rows0, rows1,
                  sem0, sem1):
        wid = lax.axis_index("s") * nc + lax.axis_index("c")
        base = pl.multiple_of(wid * per_w, ch)
        pltpu.sync_copy(idx_hbm.at[pl.ds(base, per_w)], idx_v)
        bufs = (rows0, rows1)
        sems = (sem0, sem1)

        def copy(i, slot):
            src = table_hbm.at[idx_v.at[pl.ds(pl.multiple_of(i * ch, ch), ch)]]
            return pltpu.make_async_copy(src, bufs[slot], sems[slot])

        copy(0, 0).start()
        copy(1, 1).start()

        def body(g, carry):
            i0 = g * 2
            for s in range(2):
                i = i0 + s
                copy(i, s).wait()
                pltpu.sync_copy(
                    bufs[s],
                    out_hbm.at[pl.ds(pl.multiple_of(base + i * ch, ch), ch)])

                @pl.when(i + 2 < n_ch)
                def _():
                    copy(i + 2, s).start()
            return carry

        lax.fori_loop(0, n_ch // 2, body, 0)

    return sc_gather


def _combine_kernel(g_ref, w_ref, y1_ref, mx_ref, mn_ref, s1_ref, s2_ref):
    wv = w_ref[0]
    y1 = y1_ref[0]
    mx = None
    mn = None
    s1 = jnp.zeros((1, y1.shape[1]), jnp.float32)
    s2 = jnp.zeros((1, y1.shape[1]), jnp.float32)
    for j in range(K):
        g = g_ref[0, :, j, :]
        o = y1 + wv[:, j:j + 1] * g
        mx = o if mx is None else jnp.maximum(mx, o)
        mn = o if mn is None else jnp.minimum(mn, o)
        s1 = s1 + jnp.sum(o, axis=0, keepdims=True)
        s2 = s2 + jnp.sum(o * o, axis=0, keepdims=True)
    mx_ref[0] = mx
    mn_ref[0] = mn
    step = pl.program_id(0) * pl.num_programs(1) + pl.program_id(1)

    @pl.when(step == 0)
    def _():
        s1_ref[...] = jnp.zeros_like(s1_ref)
        s2_ref[...] = jnp.zeros_like(s2_ref)

    s1_ref[...] += s1
    s2_ref[...] += s2


def _finalize_kernel(mx_ref, mn_ref, s1_ref, s2_ref, g_ref, be_ref, o_ref,
                     *, count):
    s1 = s1_ref[...]
    s2 = s2_ref[...]
    mean = s1 * (1.0 / count)
    var = s2 * (1.0 / count) - mean * mean
    gm = g_ref[...]
    scale = gm * lax.rsqrt(var + 1e-5)
    shift = be_ref[...] - mean * scale
    sel = jnp.where(gm >= 0.0, mx_ref[0], mn_ref[0])
    o_ref[0] = jnp.maximum(sel * scale + shift, 0.0)


def kernel(x, W, b, gamma, beta):
    B, Fin, N = x.shape
    Fout = W.shape[0]
    xt = jnp.transpose(x, (0, 2, 1))
    w1t = jnp.transpose(W[:, :Fin])
    w2t = jnp.transpose(W[:, Fin:])
    b2 = b.reshape(1, Fout)
    g2 = gamma.reshape(1, Fout)
    be2 = beta.reshape(1, Fout)

    tn = TOPK_TN
    idx, wgt, y1, y2 = pl.pallas_call(
        _topk_kernel,
        grid=(B, N // tn),
        in_specs=[
            pl.BlockSpec((1, tn, Fin), lambda bb, i: (bb, i, 0)),
            pl.BlockSpec((1, Fin, N), lambda bb, i: (bb, 0, 0)),
            pl.BlockSpec((Fin, Fout), lambda bb, i: (0, 0)),
            pl.BlockSpec((Fin, Fout), lambda bb, i: (0, 0)),
            pl.BlockSpec((1, Fout), lambda bb, i: (0, 0)),
        ],
        out_specs=[
            pl.BlockSpec((1, tn, K), lambda bb, i: (bb, i, 0)),
            pl.BlockSpec((1, tn, K), lambda bb, i: (bb, i, 0)),
            pl.BlockSpec((1, tn, Fout), lambda bb, i: (bb, i, 0)),
            pl.BlockSpec((1, tn, Fout), lambda bb, i: (bb, i, 0)),
        ],
        out_shape=[
            jax.ShapeDtypeStruct((B, N, K), jnp.int32),
            jax.ShapeDtypeStruct((B, N, K), jnp.float32),
            jax.ShapeDtypeStruct((B, N, Fout), jnp.float32),
            jax.ShapeDtypeStruct((B, N, Fout), jnp.float32),
        ],
    )(xt, x, w1t, w2t, b2)

    rows_total = B * N * K
    gath = _make_sc_gather(rows_total, Fout)(
        y2.reshape(B * N, Fout), idx.reshape(rows_total))
    gath = gath.reshape(B, N, K, Fout)

    ctn = COMB_TN
    mx, mn, s1, s2 = pl.pallas_call(
        _combine_kernel,
        grid=(B, N // ctn),
        in_specs=[
            pl.BlockSpec((1, ctn, K, Fout), lambda bb, i: (bb, i, 0, 0)),
            pl.BlockSpec((1, ctn, K), lambda bb, i: (bb, i, 0)),
            pl.BlockSpec((1, ctn, Fout), lambda bb, i: (bb, i, 0)),
        ],
        out_specs=[
            pl.BlockSpec((1, ctn, Fout), lambda bb, i: (bb, i, 0)),
            pl.BlockSpec((1, ctn, Fout), lambda bb, i: (bb, i, 0)),
            pl.BlockSpec((1, Fout), lambda bb, i: (0, 0)),
            pl.BlockSpec((1, Fout), lambda bb, i: (0, 0)),
        ],
        out_shape=[
            jax.ShapeDtypeStruct((B, N, Fout), jnp.float32),
            jax.ShapeDtypeStruct((B, N, Fout), jnp.float32),
            jax.ShapeDtypeStruct((1, Fout), jnp.float32),
            jax.ShapeDtypeStruct((1, Fout), jnp.float32),
        ],
    )(gath, wgt, y1)

    fin_k = functools.partial(_finalize_kernel, count=float(B * N * K))
    out = pl.pallas_call(
        fin_k,
        grid=(B, N // tn),
        in_specs=[
            pl.BlockSpec((1, tn, Fout), lambda bb, i: (bb, i, 0)),
            pl.BlockSpec((1, tn, Fout), lambda bb, i: (bb, i, 0)),
            pl.BlockSpec((1, Fout), lambda bb, i: (0, 0)),
            pl.BlockSpec((1, Fout), lambda bb, i: (0, 0)),
            pl.BlockSpec((1, Fout), lambda bb, i: (0, 0)),
            pl.BlockSpec((1, Fout), lambda bb, i: (0, 0)),
        ],
        out_specs=pl.BlockSpec((1, tn, Fout), lambda bb, i: (bb, i, 0)),
        out_shape=jax.ShapeDtypeStruct((B, N, Fout), jnp.float32),
    )(mx, mn, s1, s2, g2, be2)

    return jnp.transpose(out, (0, 2, 1))


# speedup vs baseline: 16.8709x; 1.2213x over previous
import functools

import jax
import jax.numpy as jnp
from jax import lax
from jax.experimental import pallas as pl
from jax.experimental.pallas import tpu as pltpu
from jax.experimental.pallas import tpu_sc as plsc

K = 20
TOPK_TN = 256
COMB_TN = 128
GATHER_CHUNK = 128


def _topk_kernel(xt_ref, x_ref, w1_ref, w2_ref, b_ref,
                 idx_ref, w_ref, y1_ref, y2_ref, scol_ref):
    xt = xt_ref[0]
    xb = x_ref[0]

    @pl.when(pl.program_id(1) == 0)
    def _():
        scol_ref[...] = jnp.sum(xb * xb, axis=0, keepdims=True)

    s_row = jnp.sum(xt * xt, axis=1, keepdims=True)
    s_col = scol_ref[...]
    prod = jnp.dot(xt, xb, preferred_element_type=jnp.float32,
                   precision=lax.Precision.HIGHEST)
    d = (-2.0 * prod + s_row) + s_col
    d = jnp.maximum(d, 0.0)
    tn, n = d.shape
    col = lax.broadcasted_iota(jnp.int32, (tn, n), 1)
    off = pl.program_id(0) * jnp.int32(n)
    bits = lax.bitcast_convert_type(d, jnp.int32)
    key_i = jnp.bitwise_or(jnp.bitwise_and(bits, jnp.int32(~2047)), col)
    key = lax.bitcast_convert_type(key_i, jnp.float32)
    idx_cols = []
    w_cols = []
    for t in range(K + 1):
        mf = jnp.min(key, axis=1, keepdims=True)
        if t > 0:
            mk = lax.bitcast_convert_type(mf, jnp.int32)
            idx_cols.append(jnp.bitwise_and(mk, jnp.int32(2047)))
            m = lax.bitcast_convert_type(
                jnp.bitwise_and(mk, jnp.int32(~2047)), jnp.float32)
            dist_sat = 2.0 * jax.nn.sigmoid(m) - 1.0
            w_cols.append(1.0 - dist_sat)
        key = jnp.where(key == mf, jnp.float32(jnp.inf), key)
    idx_ref[0] = jnp.concatenate(idx_cols, axis=1) + off
    w_ref[0] = jnp.concatenate(w_cols, axis=1)
    y1_ref[0] = jnp.dot(xt, w1_ref[...], preferred_element_type=jnp.float32,
                        precision=lax.Precision.HIGHEST) + b_ref[...]
    y2_ref[0] = jnp.dot(xt, w2_ref[...], preferred_element_type=jnp.float32,
                        precision=lax.Precision.HIGHEST)


def _make_sc_gather(rows_total, d_model):
    info = plsc.get_sparse_core_info()
    nc, ns = info.num_cores, info.num_subcores
    nw = nc * ns
    per_w = rows_total // nw
    ch = GATHER_CHUNK
    n_ch = per_w // ch
    assert per_w % ch == 0 and n_ch % 2 == 0
    mesh = plsc.VectorSubcoreMesh(core_axis_name="c", subcore_axis_name="s")

    @functools.partial(
        pl.kernel, mesh=mesh,
        compiler_params=pltpu.CompilerParams(use_tc_tiling_on_sc=False),
        out_type=jax.ShapeDtypeStruct((rows_total, d_model), jnp.float32),
        scratch_types=[
            pltpu.VMEM((per_w,), jnp.int32),
            pltpu.VMEM((ch, d_model), jnp.float32),
            pltpu.VMEM((ch, d_model), jnp.float32),
            pltpu.SemaphoreType.DMA,
            pltpu.SemaphoreType.DMA,
        ],
    )
    def sc_gather(table_hbm, idx_hbm, out_hbm, idx_v, rows0, rows1,
                  sem0, sem1):
        wid = lax.axis_index("s") * nc + lax.axis_index("c")
        base = pl.multiple_of(wid * per_w, ch)
        pltpu.sync_copy(idx_hbm.at[pl.ds(base, per_w)], idx_v)
        bufs = (rows0, rows1)
        sems = (sem0, sem1)

        def copy(i, slot):
            src = table_hbm.at[idx_v.at[pl.ds(pl.multiple_of(i * ch, ch), ch)]]
            return pltpu.make_async_copy(src, bufs[slot], sems[slot])

        copy(0, 0).start()
        copy(1, 1).start()

        def body(g, carry):
            i0 = g * 2
            for s in range(2):
                i = i0 + s
                copy(i, s).wait()
                pltpu.sync_copy(
                    bufs[s],
                    out_hbm.at[pl.ds(pl.multiple_of(base + i * ch, ch), ch)])

                @pl.when(i + 2 < n_ch)
                def _():
                    copy(i + 2, s).start()
            return carry

        lax.fori_loop(0, n_ch // 2, body, 0)

    return sc_gather


def _combine_kernel(g_ref, w_ref, y1_ref, mx_ref, mn_ref, s1_ref, s2_ref):
    wv = w_ref[0]
    y1 = y1_ref[0]
    mx = None
    mn = None
    s1 = jnp.zeros((1, y1.shape[1]), jnp.float32)
    s2 = jnp.zeros((1, y1.shape[1]), jnp.float32)
    for j in range(K):
        g = g_ref[0, :, j, :]
        o = y1 + wv[:, j:j + 1] * g
        mx = o if mx is None else jnp.maximum(mx, o)
        mn = o if mn is None else jnp.minimum(mn, o)
        s1 = s1 + jnp.sum(o, axis=0, keepdims=True)
        s2 = s2 + jnp.sum(o * o, axis=0, keepdims=True)
    mx_ref[0] = mx
    mn_ref[0] = mn
    step = pl.program_id(0) * pl.num_programs(1) + pl.program_id(1)

    @pl.when(step == 0)
    def _():
        s1_ref[...] = jnp.zeros_like(s1_ref)
        s2_ref[...] = jnp.zeros_like(s2_ref)

    s1_ref[...] += s1
    s2_ref[...] += s2


def _finalize_kernel(mx_ref, mn_ref, s1_ref, s2_ref, g_ref, be_ref, o_ref,
                     *, count):
    s1 = s1_ref[...]
    s2 = s2_ref[...]
    mean = s1 * (1.0 / count)
    var = s2 * (1.0 / count) - mean * mean
    gm = g_ref[...]
    scale = gm * lax.rsqrt(var + 1e-5)
    shift = be_ref[...] - mean * scale
    sel = jnp.where(gm >= 0.0, mx_ref[0], mn_ref[0])
    o_ref[0] = jnp.maximum(sel * scale + shift, 0.0)


def kernel(x, W, b, gamma, beta):
    B, Fin, N = x.shape
    Fout = W.shape[0]
    xt = jnp.transpose(x, (0, 2, 1))
    w1t = jnp.transpose(W[:, :Fin])
    w2t = jnp.transpose(W[:, Fin:])
    b2 = b.reshape(1, Fout)
    g2 = gamma.reshape(1, Fout)
    be2 = beta.reshape(1, Fout)

    tn = TOPK_TN
    idx, wgt, y1, y2 = pl.pallas_call(
        _topk_kernel,
        grid=(B, N // tn),
        in_specs=[
            pl.BlockSpec((1, tn, Fin), lambda bb, i: (bb, i, 0)),
            pl.BlockSpec((1, Fin, N), lambda bb, i: (bb, 0, 0)),
            pl.BlockSpec((Fin, Fout), lambda bb, i: (0, 0)),
            pl.BlockSpec((Fin, Fout), lambda bb, i: (0, 0)),
            pl.BlockSpec((1, Fout), lambda bb, i: (0, 0)),
        ],
        out_specs=[
            pl.BlockSpec((1, tn, K), lambda bb, i: (bb, i, 0)),
            pl.BlockSpec((1, tn, K), lambda bb, i: (bb, i, 0)),
            pl.BlockSpec((1, tn, Fout), lambda bb, i: (bb, i, 0)),
            pl.BlockSpec((1, tn, Fout), lambda bb, i: (bb, i, 0)),
        ],
        out_shape=[
            jax.ShapeDtypeStruct((B, N, K), jnp.int32),
            jax.ShapeDtypeStruct((B, N, K), jnp.float32),
            jax.ShapeDtypeStruct((B, N, Fout), jnp.float32),
            jax.ShapeDtypeStruct((B, N, Fout), jnp.float32),
        ],
        scratch_shapes=[pltpu.VMEM((1, N), jnp.float32)],
    )(xt, x, w1t, w2t, b2)

    rows_total = B * N * K
    gath = _make_sc_gather(rows_total, Fout)(
        y2.reshape(B * N, Fout), idx.reshape(rows_total))
    gath = gath.reshape(B, N, K, Fout)

    ctn = COMB_TN
    mx, mn, s1, s2 = pl.pallas_call(
        _combine_kernel,
        grid=(B, N // ctn),
        in_specs=[
            pl.BlockSpec((1, ctn, K, Fout), lambda bb, i: (bb, i, 0, 0)),
            pl.BlockSpec((1, ctn, K), lambda bb, i: (bb, i, 0)),
            pl.BlockSpec((1, ctn, Fout), lambda bb, i: (bb, i, 0)),
        ],
        out_specs=[
            pl.BlockSpec((1, ctn, Fout), lambda bb, i: (bb, i, 0)),
            pl.BlockSpec((1, ctn, Fout), lambda bb, i: (bb, i, 0)),
            pl.BlockSpec((1, Fout), lambda bb, i: (0, 0)),
            pl.BlockSpec((1, Fout), lambda bb, i: (0, 0)),
        ],
        out_shape=[
            jax.ShapeDtypeStruct((B, N, Fout), jnp.float32),
            jax.ShapeDtypeStruct((B, N, Fout), jnp.float32),
            jax.ShapeDtypeStruct((1, Fout), jnp.float32),
            jax.ShapeDtypeStruct((1, Fout), jnp.float32),
        ],
    )(gath, wgt, y1)

    fin_k = functools.partial(_finalize_kernel, count=float(B * N * K))
    out = pl.pallas_call(
        fin_k,
        grid=(B, N // tn),
        in_specs=[
            pl.BlockSpec((1, tn, Fout), lambda bb, i: (bb, i, 0)),
            pl.BlockSpec((1, tn, Fout), lambda bb, i: (bb, i, 0)),
            pl.BlockSpec((1, Fout), lambda bb, i: (0, 0)),
            pl.BlockSpec((1, Fout), lambda bb, i: (0, 0)),
            pl.BlockSpec((1, Fout), lambda bb, i: (0, 0)),
            pl.BlockSpec((1, Fout), lambda bb, i: (0, 0)),
        ],
        out_specs=pl.BlockSpec((1, tn, Fout), lambda bb, i: (bb, i, 0)),
        out_shape=jax.ShapeDtypeStruct((B, N, Fout), jnp.float32),
    )(mx, mn, s1, s2, g2, be2)

    return jnp.transpose(out, (0, 2, 1))


# speedup vs baseline: 16.9406x; 1.0041x over previous
import functools

import jax
import jax.numpy as jnp
from jax import lax
from jax.experimental import pallas as pl
from jax.experimental.pallas import tpu as pltpu
from jax.experimental.pallas import tpu_sc as plsc

K = 20
TOPK_TN = 256
COMB_TN = 128
GATHER_CHUNK = 128


def _topk_kernel(xt_ref, x_ref, w1_ref, w2_ref, b_ref,
                 idx_ref, w_ref, y1_ref, y2_ref, scol_ref):
    xt = xt_ref[0]
    xb = x_ref[0]

    @pl.when(pl.program_id(1) == 0)
    def _():
        scol_ref[...] = jnp.sum(xb * xb, axis=0, keepdims=True)

    s_row = jnp.sum(xt * xt, axis=1, keepdims=True)
    s_col = scol_ref[...]
    prod = jnp.dot(xt, xb, preferred_element_type=jnp.float32,
                   precision=lax.Precision.HIGHEST)
    d = (-2.0 * prod + s_row) + s_col
    d = jnp.maximum(d, 0.0)
    tn, n = d.shape
    col = lax.broadcasted_iota(jnp.int32, (tn, n), 1)
    off = pl.program_id(0) * jnp.int32(n)
    bits = lax.bitcast_convert_type(d, jnp.int32)
    key_i = jnp.bitwise_or(jnp.bitwise_and(bits, jnp.int32(~2047)), col)
    key = lax.bitcast_convert_type(key_i, jnp.float32)
    idx_cols = []
    w_cols = []
    for t in range(K + 1):
        mf = jnp.min(key, axis=1, keepdims=True)
        if t > 0:
            mk = lax.bitcast_convert_type(mf, jnp.int32)
            idx_cols.append(jnp.bitwise_and(mk, jnp.int32(2047)))
            m = lax.bitcast_convert_type(
                jnp.bitwise_and(mk, jnp.int32(~2047)), jnp.float32)
            dist_sat = 2.0 * jax.nn.sigmoid(m) - 1.0
            w_cols.append(1.0 - dist_sat)
        key = jnp.where(key == mf, jnp.float32(jnp.inf), key)
    idx_ref[0] = jnp.concatenate(idx_cols, axis=1) + off
    w_ref[0] = jnp.concatenate(w_cols, axis=1)
    y1_ref[0] = jnp.dot(xt, w1_ref[...], preferred_element_type=jnp.float32,
                        precision=lax.Precision.HIGHEST) + b_ref[...]
    y2_ref[0] = jnp.dot(xt, w2_ref[...], preferred_element_type=jnp.float32,
                        precision=lax.Precision.HIGHEST)


def _make_sc_gather(rows_total, d_model):
    info = plsc.get_sparse_core_info()
    nc, ns = info.num_cores, info.num_subcores
    nw = nc * ns
    per_w = rows_total // nw
    ch = GATHER_CHUNK
    n_ch = per_w // ch
    assert per_w % ch == 0 and n_ch % 4 == 0 and n_ch >= 8
    mesh = plsc.VectorSubcoreMesh(core_axis_name="c", subcore_axis_name="s")

    @functools.partial(
        pl.kernel, mesh=mesh,
        compiler_params=pltpu.CompilerParams(use_tc_tiling_on_sc=False),
        out_type=jax.ShapeDtypeStruct((rows_total, d_model), jnp.float32),
        scratch_types=[
            pltpu.VMEM((per_w,), jnp.int32),
            pltpu.VMEM((ch, d_model), jnp.float32),
            pltpu.VMEM((ch, d_model), jnp.float32),
            pltpu.VMEM((ch, d_model), jnp.float32),
            pltpu.VMEM((ch, d_model), jnp.float32),
            pltpu.SemaphoreType.DMA,
            pltpu.SemaphoreType.DMA,
            pltpu.SemaphoreType.DMA,
            pltpu.SemaphoreType.DMA,
            pltpu.SemaphoreType.DMA,
            pltpu.SemaphoreType.DMA,
            pltpu.SemaphoreType.DMA,
            pltpu.SemaphoreType.DMA,
        ],
    )
    def sc_gather(table_hbm, idx_hbm, out_hbm, idx_v,
                  rows0, rows1, rows2, rows3,
                  gs0, gs1, gs2, gs3, ws0, ws1, ws2, ws3):
        wid = lax.axis_index("s") * nc + lax.axis_index("c")
        base = pl.multiple_of(wid * per_w, ch)
        pltpu.sync_copy(idx_hbm.at[pl.ds(base, per_w)], idx_v)
        bufs = (rows0, rows1, rows2, rows3)
        gsems = (gs0, gs1, gs2, gs3)
        wsems = (ws0, ws1, ws2, ws3)

        def gather(i, slot):
            src = table_hbm.at[idx_v.at[pl.ds(pl.multiple_of(i * ch, ch), ch)]]
            return pltpu.make_async_copy(src, bufs[slot], gsems[slot])

        def write(i, slot):
            dst = out_hbm.at[pl.ds(pl.multiple_of(base + i * ch, ch), ch)]
            return pltpu.make_async_copy(bufs[slot], dst, wsems[slot])

        for s in range(4):
            gather(s, s).start()

        def body(g, carry):
            i0 = g * 4
            for s in range(4):
                i = i0 + s
                gather(i, s).wait()
                write(i, s).start()

                @pl.when(i + 4 < n_ch)
                def _():
                    write(i, s).wait()
                    gather(i + 4, s).start()
            return carry

        lax.fori_loop(0, n_ch // 4, body, 0)
        for s in range(4):
            write(n_ch - 4 + s, s).wait()

    return sc_gather


def _combine_kernel(g_ref, w_ref, y1_ref, mx_ref, mn_ref, s1_ref, s2_ref):
    wv = w_ref[0]
    y1 = y1_ref[0]
    mx = None
    mn = None
    s1 = jnp.zeros((1, y1.shape[1]), jnp.float32)
    s2 = jnp.zeros((1, y1.shape[1]), jnp.float32)
    for j in range(K):
        g = g_ref[0, :, j, :]
        o = y1 + wv[:, j:j + 1] * g
        mx = o if mx is None else jnp.maximum(mx, o)
        mn = o if mn is None else jnp.minimum(mn, o)
        s1 = s1 + jnp.sum(o, axis=0, keepdims=True)
        s2 = s2 + jnp.sum(o * o, axis=0, keepdims=True)
    mx_ref[0] = mx
    mn_ref[0] = mn
    step = pl.program_id(0) * pl.num_programs(1) + pl.program_id(1)

    @pl.when(step == 0)
    def _():
        s1_ref[...] = jnp.zeros_like(s1_ref)
        s2_ref[...] = jnp.zeros_like(s2_ref)

    s1_ref[...] += s1
    s2_ref[...] += s2


def _finalize_kernel(mx_ref, mn_ref, s1_ref, s2_ref, g_ref, be_ref, o_ref,
                     *, count):
    s1 = s1_ref[...]
    s2 = s2_ref[...]
    mean = s1 * (1.0 / count)
    var = s2 * (1.0 / count) - mean * mean
    gm = g_ref[...]
    scale = gm * lax.rsqrt(var + 1e-5)
    shift = be_ref[...] - mean * scale
    sel = jnp.where(gm >= 0.0, mx_ref[0], mn_ref[0])
    o_ref[0] = jnp.maximum(sel * scale + shift, 0.0)


def kernel(x, W, b, gamma, beta):
    B, Fin, N = x.shape
    Fout = W.shape[0]
    xt = jnp.transpose(x, (0, 2, 1))
    w1t = jnp.transpose(W[:, :Fin])
    w2t = jnp.transpose(W[:, Fin:])
    b2 = b.reshape(1, Fout)
    g2 = gamma.reshape(1, Fout)
    be2 = beta.reshape(1, Fout)

    tn = TOPK_TN
    idx, wgt, y1, y2 = pl.pallas_call(
        _topk_kernel,
        grid=(B, N // tn),
        in_specs=[
            pl.BlockSpec((1, tn, Fin), lambda bb, i: (bb, i, 0)),
            pl.BlockSpec((1, Fin, N), lambda bb, i: (bb, 0, 0)),
            pl.BlockSpec((Fin, Fout), lambda bb, i: (0, 0)),
            pl.BlockSpec((Fin, Fout), lambda bb, i: (0, 0)),
            pl.BlockSpec((1, Fout), lambda bb, i: (0, 0)),
        ],
        out_specs=[
            pl.BlockSpec((1, tn, K), lambda bb, i: (bb, i, 0)),
            pl.BlockSpec((1, tn, K), lambda bb, i: (bb, i, 0)),
            pl.BlockSpec((1, tn, Fout), lambda bb, i: (bb, i, 0)),
            pl.BlockSpec((1, tn, Fout), lambda bb, i: (bb, i, 0)),
        ],
        out_shape=[
            jax.ShapeDtypeStruct((B, N, K), jnp.int32),
            jax.ShapeDtypeStruct((B, N, K), jnp.float32),
            jax.ShapeDtypeStruct((B, N, Fout), jnp.float32),
            jax.ShapeDtypeStruct((B, N, Fout), jnp.float32),
        ],
        scratch_shapes=[pltpu.VMEM((1, N), jnp.float32)],
    )(xt, x, w1t, w2t, b2)

    rows_total = B * N * K
    gath = _make_sc_gather(rows_total, Fout)(
        y2.reshape(B * N, Fout), idx.reshape(rows_total))
    gath = gath.reshape(B, N, K, Fout)

    ctn = COMB_TN
    mx, mn, s1, s2 = pl.pallas_call(
        _combine_kernel,
        grid=(B, N // ctn),
        in_specs=[
            pl.BlockSpec((1, ctn, K, Fout), lambda bb, i: (bb, i, 0, 0)),
            pl.BlockSpec((1, ctn, K), lambda bb, i: (bb, i, 0)),
            pl.BlockSpec((1, ctn, Fout), lambda bb, i: (bb, i, 0)),
        ],
        out_specs=[
            pl.BlockSpec((1, ctn, Fout), lambda bb, i: (bb, i, 0)),
            pl.BlockSpec((1, ctn, Fout), lambda bb, i: (bb, i, 0)),
            pl.BlockSpec((1, Fout), lambda bb, i: (0, 0)),
            pl.BlockSpec((1, Fout), lambda bb, i: (0, 0)),
        ],
        out_shape=[
            jax.ShapeDtypeStruct((B, N, Fout), jnp.float32),
            jax.ShapeDtypeStruct((B, N, Fout), jnp.float32),
            jax.ShapeDtypeStruct((1, Fout), jnp.float32),
            jax.ShapeDtypeStruct((1, Fout), jnp.float32),
        ],
    )(gath, wgt, y1)

    fin_k = functools.partial(_finalize_kernel, count=float(B * N * K))
    out = pl.pallas_call(
        fin_k,
        grid=(B, N // tn),
        in_specs=[
            pl.BlockSpec((1, tn, Fout), lambda bb, i: (bb, i, 0)),
            pl.BlockSpec((1, tn, Fout), lambda bb, i: (bb, i, 0)),
            pl.BlockSpec((1, Fout), lambda bb, i: (0, 0)),
            pl.BlockSpec((1, Fout), lambda bb, i: (0, 0)),
            pl.BlockSpec((1, Fout), lambda bb, i: (0, 0)),
            pl.BlockSpec((1, Fout), lambda bb, i: (0, 0)),
        ],
        out_specs=pl.BlockSpec((1, tn, Fout), lambda bb, i: (bb, i, 0)),
        out_shape=jax.ShapeDtypeStruct((B, N, Fout), jnp.float32),
    )(mx, mn, s1, s2, g2, be2)

    return jnp.transpose(out, (0, 2, 1))
